# Initial kernel scaffold; baseline (speedup 1.0000x reference)
import jax
import jax.numpy as jnp
from jax.experimental import pallas as pl


def kernel(x, edge_attr, coords, bc_disp, bc_rot, edge_index, params):
    raise NotImplementedError("write your pallas kernel here")


# speedup vs baseline: 2.7863x; 2.7863x over previous
import functools

import jax
import jax.numpy as jnp
from jax import lax
from jax.experimental import pallas as pl
from jax.experimental.pallas import tpu as pltpu
from jax.experimental.pallas import tpu_sc as plsc

N = 10000
E = 160000
H = 128

EB = 2000
N_EDGE_BLOCKS = E // EB


def _mlp2_body(x_ref, w1_ref, b1_ref, w2_ref, b2_ref, o_ref):
    t = jnp.dot(x_ref[...], w1_ref[...], preferred_element_type=jnp.float32)
    t = jax.nn.relu(t + b1_ref[...])
    o = jnp.dot(t, w2_ref[...], preferred_element_type=jnp.float32)
    o_ref[...] = o + b2_ref[...]


def _node_encode(x, w1, b1, w2, b2):
    return pl.pallas_call(
        _mlp2_body,
        out_shape=jax.ShapeDtypeStruct((N, H), jnp.float32),
    )(x, w1, b1, w2, b2)


def _edge_prep_body(ea_ref, w1_ref, b1_ref, w2_ref, b2_ref,
                    we_ref, be_ref, ec0_ref, ec1_ref, ec2_ref):
    t = jnp.dot(ea_ref[...], w1_ref[...], preferred_element_type=jnp.float32)
    t = jax.nn.relu(t + b1_ref[...])
    e = jnp.dot(t, w2_ref[...], preferred_element_type=jnp.float32) + b2_ref[...]
    for l, o_ref in enumerate((ec0_ref, ec1_ref, ec2_ref)):
        we = we_ref[l]
        ec = jnp.dot(e, we, preferred_element_type=jnp.float32)
        o_ref[...] = ec + be_ref[l]


def _edge_prep(edge_attr, w1, b1, w2, b2, we_all, be_all):
    blk = lambda i: (i, 0)
    zero = lambda i: (0, 0)
    out_specs = [pl.BlockSpec((EB, H), blk)] * 3
    return pl.pallas_call(
        _edge_prep_body,
        grid=(N_EDGE_BLOCKS,),
        in_specs=[
            pl.BlockSpec((EB, 10), blk),
            pl.BlockSpec((10, H), zero),
            pl.BlockSpec((1, H), zero),
            pl.BlockSpec((H, H), zero),
            pl.BlockSpec((1, H), zero),
            pl.BlockSpec((3, H, H), lambda i: (0, 0, 0)),
            pl.BlockSpec((3, 1, H), lambda i: (0, 0, 0)),
        ],
        out_specs=out_specs,
        out_shape=[jax.ShapeDtypeStruct((E, H), jnp.float32)] * 3,
    )(edge_attr, w1, b1, w2, b2, we_all, be_all)


def _ab_body(h_ref, c_ref, wa_ref, wb_ref, wr_ref, a_ref, b_ref):
    cr = jnp.dot(c_ref[...], wr_ref[...], preferred_element_type=jnp.float32)
    a_ref[...] = jnp.dot(h_ref[...], wa_ref[...],
                         preferred_element_type=jnp.float32) - cr
    b_ref[...] = jnp.dot(h_ref[...], wb_ref[...],
                         preferred_element_type=jnp.float32) + cr


def _ab_tables(h, coords, wa, wb, wr):
    return pl.pallas_call(
        _ab_body,
        out_shape=[jax.ShapeDtypeStruct((N, H), jnp.float32)] * 2,
    )(h, coords, wa, wb, wr)


def _msg_body(g1_ref, g2_ref, ec_ref, w2_ref, b2_ref, o_ref):
    hidden = jax.nn.relu(g1_ref[...] + g2_ref[...] + ec_ref[...])
    o = jnp.dot(hidden, w2_ref[...], preferred_element_type=jnp.float32)
    o_ref[...] = o + b2_ref[...]


def _messages(g1, g2, ec, w2, b2):
    blk = lambda i: (i, 0)
    zero = lambda i: (0, 0)
    return pl.pallas_call(
        _msg_body,
        grid=(N_EDGE_BLOCKS,),
        in_specs=[
            pl.BlockSpec((EB, H), blk),
            pl.BlockSpec((EB, H), blk),
            pl.BlockSpec((EB, H), blk),
            pl.BlockSpec((H, H), zero),
            pl.BlockSpec((1, H), zero),
        ],
        out_specs=pl.BlockSpec((EB, H), blk),
        out_shape=jax.ShapeDtypeStruct((E, H), jnp.float32),
    )(g1, g2, ec, w2, b2)


def _update_body(h_ref, p0_ref, p1_ref, wh_ref, wa_ref, b1_ref,
                 w2_ref, b2_ref, g_ref, bln_ref, o_ref):
    h = h_ref[...]
    aggr = p0_ref[...] + p1_ref[...]
    t = jnp.dot(h, wh_ref[...], preferred_element_type=jnp.float32)
    t = t + jnp.dot(aggr, wa_ref[...], preferred_element_type=jnp.float32)
    t = jax.nn.relu(t + b1_ref[...])
    out = jnp.dot(t, w2_ref[...], preferred_element_type=jnp.float32) + b2_ref[...]
    y = h + out
    mu = jnp.mean(y, axis=-1, keepdims=True)
    var = jnp.mean((y - mu) ** 2, axis=-1, keepdims=True)
    o_ref[...] = g_ref[...] * (y - mu) * lax.rsqrt(var + 1e-5) + bln_ref[...]


def _update(h, p0, p1, wh, wa, b1, w2, b2, g, b):
    return pl.pallas_call(
        _update_body,
        out_shape=jax.ShapeDtypeStruct((N, H), jnp.float32),
    )(h, p0, p1, wh, wa, b1, w2, b2, g, b)


def _decoder_body(c_ref, h_ref, bcd_ref, bcr_ref, wc_ref, wh_ref, b1_ref,
                  w2_ref, b2_ref, w3_ref, b3_ref, w4_ref, b4_ref, o_ref):
    c = c_ref[...]
    c_min = jnp.min(c, axis=0, keepdims=True)
    c_max = jnp.max(c, axis=0, keepdims=True)
    c_range = jnp.clip(c_max - c_min, 1e-8, None)
    cn = (c - c_min) / c_range
    t = jnp.dot(cn, wc_ref[...], preferred_element_type=jnp.float32)
    t = t + jnp.dot(h_ref[...], wh_ref[...], preferred_element_type=jnp.float32)
    t = jax.nn.relu(t + b1_ref[...])
    t = jnp.dot(t, w2_ref[...], preferred_element_type=jnp.float32) + b2_ref[...]
    t = jax.nn.relu(t)
    t = jnp.dot(t, w3_ref[...], preferred_element_type=jnp.float32) + b3_ref[...]
    t = jax.nn.relu(t)
    pred = jnp.dot(t, w4_ref[...], preferred_element_type=jnp.float32) + b4_ref[...]
    mask01 = 1.0 - bcd_ref[...]
    mask2 = 1.0 - bcr_ref[...]
    col = lax.broadcasted_iota(jnp.int32, pred.shape, 1)
    o_ref[...] = pred * jnp.where(col < 2, mask01, mask2)


def _decoder(coords, h, bc_disp, bc_rot, wc, wh, b1, w2, b2, w3, b3, w4, b4):
    return pl.pallas_call(
        _decoder_body,
        out_shape=jax.ShapeDtypeStruct((N, 3), jnp.float32),
    )(coords, h, bc_disp, bc_rot, wc, wh, b1, w2, b2, w3, b3, w4, b4)


_NC = 2
_NS = 16
_NW = _NC * _NS
_PER_W = E // _NW
_CHUNK = 128
_NFULL = _PER_W // _CHUNK
_TAIL = _PER_W - _NFULL * _CHUNK


def _sc_gather2(a_tab, b_tab, src, dst):
    mesh = plsc.VectorSubcoreMesh(core_axis_name="c", subcore_axis_name="s")

    @functools.partial(
        pl.kernel,
        mesh=mesh,
        out_type=[jax.ShapeDtypeStruct((E, H), jnp.float32)] * 2,
        scratch_types=[
            pltpu.VMEM((_CHUNK,), jnp.int32),
            pltpu.VMEM((_CHUNK,), jnp.int32),
            pltpu.VMEM((_CHUNK, H), jnp.float32),
            pltpu.VMEM((_CHUNK, H), jnp.float32),
            pltpu.VMEM((_TAIL,), jnp.int32),
            pltpu.VMEM((_TAIL,), jnp.int32),
            pltpu.VMEM((_TAIL, H), jnp.float32),
            pltpu.VMEM((_TAIL, H), jnp.float32),
            pltpu.SemaphoreType.DMA,
            pltpu.SemaphoreType.DMA,
        ],
    )
    def k(a_hbm, b_hbm, src_hbm, dst_hbm, g1_hbm, g2_hbm,
          i1, i2, r1, r2, ti1, ti2, tr1, tr2, sem1, sem2):
        wid = lax.axis_index("s") * _NC + lax.axis_index("c")
        base = wid * _PER_W

        @pl.loop(0, _NFULL)
        def _(ci):
            off = base + ci * _CHUNK
            pltpu.sync_copy(src_hbm.at[pl.ds(off, _CHUNK)], i1)
            pltpu.sync_copy(dst_hbm.at[pl.ds(off, _CHUNK)], i2)
            c1 = pltpu.async_copy(a_hbm.at[i1], r1, sem1)
            c2 = pltpu.async_copy(b_hbm.at[i2], r2, sem2)
            c1.wait()
            c2.wait()
            pltpu.sync_copy(r1, g1_hbm.at[pl.ds(off, _CHUNK)])
            pltpu.sync_copy(r2, g2_hbm.at[pl.ds(off, _CHUNK)])

        off = base + _NFULL * _CHUNK
        pltpu.sync_copy(src_hbm.at[pl.ds(off, _TAIL)], ti1)
        pltpu.sync_copy(dst_hbm.at[pl.ds(off, _TAIL)], ti2)
        c1 = pltpu.async_copy(a_hbm.at[ti1], tr1, sem1)
        c2 = pltpu.async_copy(b_hbm.at[ti2], tr2, sem2)
        c1.wait()
        c2.wait()
        pltpu.sync_copy(tr1, g1_hbm.at[pl.ds(off, _TAIL)])
        pltpu.sync_copy(tr2, g2_hbm.at[pl.ds(off, _TAIL)])

    return k(a_tab, b_tab, src, dst)


_E_PER_CORE = E // _NC
_ROWS_PER_S = 624
_ROWS_REM = N - _NS * _ROWS_PER_S


def _sc_scatter_add(msgs, dst, zeros_nh):
    mesh = plsc.VectorSubcoreMesh(core_axis_name="c", subcore_axis_name="s")

    @functools.partial(
        pl.kernel,
        mesh=mesh,
        out_type=jax.ShapeDtypeStruct((_NC * N, H), jnp.float32),
        scratch_types=[
            pltpu.VMEM((_CHUNK,), jnp.int32),
            pltpu.VMEM((_CHUNK, H), jnp.float32),
            pltpu.VMEM((_TAIL,), jnp.int32),
            pltpu.VMEM((_TAIL, H), jnp.float32),
            pltpu.VMEM_SHARED((N, H), jnp.float32),
            pltpu.SemaphoreType.DMA,
        ],
    )
    def k(m_hbm, d_hbm, z_hbm, o_hbm, idx, rows, tidx, trows, acc, sem):
        cid = lax.axis_index("c")
        sid = lax.axis_index("s")
        r0 = sid * _ROWS_PER_S
        pltpu.sync_copy(z_hbm.at[pl.ds(r0, _ROWS_PER_S)],
                        acc.at[pl.ds(r0, _ROWS_PER_S)])

        @pl.when(sid == _NS - 1)
        def _():
            pltpu.sync_copy(z_hbm.at[pl.ds(_NS * _ROWS_PER_S, _ROWS_REM)],
                            acc.at[pl.ds(_NS * _ROWS_PER_S, _ROWS_REM)])

        plsc.subcore_barrier()

        base = cid * _E_PER_CORE + sid * _PER_W

        @pl.loop(0, _NFULL)
        def _(ci):
            off = base + ci * _CHUNK
            pltpu.sync_copy(d_hbm.at[pl.ds(off, _CHUNK)], idx)
            pltpu.sync_copy(m_hbm.at[pl.ds(off, _CHUNK)], rows)
            pltpu.sync_copy(rows, acc.at[idx], add=True)

        off = base + _NFULL * _CHUNK
        pltpu.sync_copy(d_hbm.at[pl.ds(off, _TAIL)], tidx)
        pltpu.sync_copy(m_hbm.at[pl.ds(off, _TAIL)], trows)
        pltpu.sync_copy(trows, acc.at[tidx], add=True)

        plsc.subcore_barrier()
        pltpu.sync_copy(acc.at[pl.ds(r0, _ROWS_PER_S)],
                        o_hbm.at[pl.ds(cid * N + r0, _ROWS_PER_S)])

        @pl.when(sid == _NS - 1)
        def _():
            pltpu.sync_copy(
                acc.at[pl.ds(_NS * _ROWS_PER_S, _ROWS_REM)],
                o_hbm.at[pl.ds(cid * N + _NS * _ROWS_PER_S, _ROWS_REM)])

    out = k(msgs, dst, zeros_nh)
    return out[:N], out[N:]


def kernel(x, edge_attr, coords, bc_disp, bc_rot, edge_index, params):
    r2 = lambda v: v.reshape(1, -1)
    src = edge_index[0].astype(jnp.int32)
    dst = edge_index[1].astype(jnp.int32)

    (ne_w1, ne_b1), (ne_w2, ne_b2) = params['node_encoder']
    (ee_w1, ee_b1), (ee_w2, ee_b2) = params['edge_encoder']

    h = _node_encode(x, ne_w1, r2(ne_b1), ne_w2, r2(ne_b2))

    we_all = jnp.stack([lp['message_mlp'][0][0][2 * H:3 * H] for lp in params['mp_layers']])
    be_all = jnp.stack([r2(lp['message_mlp'][0][1]) for lp in params['mp_layers']])
    ecs = _edge_prep(edge_attr, ee_w1, r2(ee_b1), ee_w2, r2(ee_b2), we_all, be_all)

    zeros_nh = jnp.zeros((N, H), jnp.float32)

    for li, lp in enumerate(params['mp_layers']):
        (w1, _b1), (w2, b2) = lp['message_mlp']
        wa, wb, wr = w1[:H], w1[H:2 * H], w1[3 * H:]
        (wu1, bu1), (wu2, bu2) = lp['update_mlp']
        wh, wag = wu1[:H], wu1[H:]
        ec = ecs[li]
        for _ in range(2):
            a_tab, b_tab = _ab_tables(h, coords, wa, wb, wr)
            g1, g2 = _sc_gather2(a_tab, b_tab, src, dst)
            msgs = _messages(g1, g2, ec, w2, r2(b2))
            p0, p1 = _sc_scatter_add(msgs, dst, zeros_nh)
            h = _update(h, p0, p1, wh, wag, r2(bu1), wu2, r2(bu2),
                        r2(lp['ln_g']), r2(lp['ln_b']))

    (d_w1, d_b1), (d_w2, d_b2), (d_w3, d_b3), (d_w4, d_b4) = params['decoder']
    return _decoder(coords, h, bc_disp, bc_rot,
                    d_w1[:3], d_w1[3:], r2(d_b1), d_w2, r2(d_b2),
                    d_w3, r2(d_b3), d_w4, r2(d_b4))


# speedup vs baseline: 3.6032x; 1.2932x over previous
import functools

import jax
import jax.numpy as jnp
from jax import lax
from jax.experimental import pallas as pl
from jax.experimental.pallas import tpu as pltpu
from jax.experimental.pallas import tpu_sc as plsc

N = 10000
E = 160000
H = 128

EB = 2000
N_EDGE_BLOCKS = E // EB


def _mlp2_body(x_ref, w1_ref, b1_ref, w2_ref, b2_ref, o_ref):
    t = jnp.dot(x_ref[...], w1_ref[...], preferred_element_type=jnp.float32)
    t = jax.nn.relu(t + b1_ref[...])
    o = jnp.dot(t, w2_ref[...], preferred_element_type=jnp.float32)
    o_ref[...] = o + b2_ref[...]


def _node_encode(x, w1, b1, w2, b2):
    return pl.pallas_call(
        _mlp2_body,
        out_shape=jax.ShapeDtypeStruct((N, H), jnp.float32),
    )(x, w1, b1, w2, b2)


def _edge_prep_body(ea_ref, w1_ref, b1_ref, w2_ref, b2_ref,
                    we_ref, be_ref, ec0_ref, ec1_ref, ec2_ref):
    t = jnp.dot(ea_ref[...], w1_ref[...], preferred_element_type=jnp.float32)
    t = jax.nn.relu(t + b1_ref[...])
    e = jnp.dot(t, w2_ref[...], preferred_element_type=jnp.float32) + b2_ref[...]
    for l, o_ref in enumerate((ec0_ref, ec1_ref, ec2_ref)):
        ec = jnp.dot(e, we_ref[l], preferred_element_type=jnp.float32)
        o_ref[...] = ec + be_ref[l]


def _edge_prep(edge_attr, w1, b1, w2, b2, we_all, be_all):
    blk = lambda i: (i, 0)
    zero = lambda i: (0, 0)
    return pl.pallas_call(
        _edge_prep_body,
        grid=(N_EDGE_BLOCKS,),
        in_specs=[
            pl.BlockSpec((EB, 10), blk),
            pl.BlockSpec((10, H), zero),
            pl.BlockSpec((1, H), zero),
            pl.BlockSpec((H, H), zero),
            pl.BlockSpec((1, H), zero),
            pl.BlockSpec((3, H, H), lambda i: (0, 0, 0)),
            pl.BlockSpec((3, 1, H), lambda i: (0, 0, 0)),
        ],
        out_specs=[pl.BlockSpec((EB, H), blk)] * 3,
        out_shape=[jax.ShapeDtypeStruct((E, H), jnp.float32)] * 3,
    )(edge_attr, w1, b1, w2, b2, we_all, be_all)


def _ab_body(h_ref, c_ref, wa_ref, wb_ref, wr_ref, a_ref, b_ref):
    cr = jnp.dot(c_ref[...], wr_ref[...], preferred_element_type=jnp.float32)
    a_ref[...] = jnp.dot(h_ref[...], wa_ref[...],
                         preferred_element_type=jnp.float32) - cr
    b_ref[...] = jnp.dot(h_ref[...], wb_ref[...],
                         preferred_element_type=jnp.float32) + cr


def _ab_tables(h, coords, wa, wb, wr):
    return pl.pallas_call(
        _ab_body,
        out_shape=[jax.ShapeDtypeStruct((N, H), jnp.float32)] * 2,
    )(h, coords, wa, wb, wr)


def _update_body(h_ref, p0_ref, p1_ref, deg_ref, w2_ref, b2_ref,
                 wh_ref, wa_ref, b1_ref, wu2_ref, bu2_ref,
                 g_ref, bln_ref, o_ref):
    h = h_ref[...]
    hsum = p0_ref[...] + p1_ref[...]
    aggr = jnp.dot(hsum, w2_ref[...], preferred_element_type=jnp.float32)
    aggr = aggr + deg_ref[...] * b2_ref[...]
    t = jnp.dot(h, wh_ref[...], preferred_element_type=jnp.float32)
    t = t + jnp.dot(aggr, wa_ref[...], preferred_element_type=jnp.float32)
    t = jax.nn.relu(t + b1_ref[...])
    out = jnp.dot(t, wu2_ref[...], preferred_element_type=jnp.float32) + bu2_ref[...]
    y = h + out
    mu = jnp.mean(y, axis=-1, keepdims=True)
    var = jnp.mean((y - mu) ** 2, axis=-1, keepdims=True)
    o_ref[...] = g_ref[...] * (y - mu) * lax.rsqrt(var + 1e-5) + bln_ref[...]


def _update(h, p0, p1, deg, w2, b2, wh, wa, b1, wu2, bu2, g, b):
    return pl.pallas_call(
        _update_body,
        out_shape=jax.ShapeDtypeStruct((N, H), jnp.float32),
    )(h, p0, p1, deg, w2, b2, wh, wa, b1, wu2, bu2, g, b)


def _decoder_body(c_ref, h_ref, bcd_ref, bcr_ref, wc_ref, wh_ref, b1_ref,
                  w2_ref, b2_ref, w3_ref, b3_ref, w4_ref, b4_ref, o_ref):
    c = c_ref[...]
    c_min = jnp.min(c, axis=0, keepdims=True)
    c_max = jnp.max(c, axis=0, keepdims=True)
    c_range = jnp.clip(c_max - c_min, 1e-8, None)
    cn = (c - c_min) / c_range
    t = jnp.dot(cn, wc_ref[...], preferred_element_type=jnp.float32)
    t = t + jnp.dot(h_ref[...], wh_ref[...], preferred_element_type=jnp.float32)
    t = jax.nn.relu(t + b1_ref[...])
    t = jnp.dot(t, w2_ref[...], preferred_element_type=jnp.float32) + b2_ref[...]
    t = jax.nn.relu(t)
    t = jnp.dot(t, w3_ref[...], preferred_element_type=jnp.float32) + b3_ref[...]
    t = jax.nn.relu(t)
    pred = jnp.dot(t, w4_ref[...], preferred_element_type=jnp.float32) + b4_ref[...]
    mask01 = 1.0 - bcd_ref[...]
    mask2 = 1.0 - bcr_ref[...]
    col = lax.broadcasted_iota(jnp.int32, pred.shape, 1)
    o_ref[...] = pred * jnp.where(col < 2, mask01, mask2)


def _decoder(coords, h, bc_disp, bc_rot, wc, wh, b1, w2, b2, w3, b3, w4, b4):
    return pl.pallas_call(
        _decoder_body,
        out_shape=jax.ShapeDtypeStruct((N, 3), jnp.float32),
    )(coords, h, bc_disp, bc_rot, wc, wh, b1, w2, b2, w3, b3, w4, b4)


_NC = 2
_NS = 16
_NW = _NC * _NS
_PER_W = E // _NW
_CHUNK = 96
_NFULL = _PER_W // _CHUNK
_TAIL = _PER_W - _NFULL * _CHUNK
_ROWS_PER_S = 624
_ROWS_REM = N - _NS * _ROWS_PER_S
_DEGW = H


def _relu_add_rows(n_rows, dst_buf, b_buf, e_buf):
    @pl.loop(0, n_rows)
    def _(r):
        for cc in range(H // 16):
            sl = (r, pl.ds(cc * 16, 16))
            v = dst_buf[sl] + b_buf[sl] + e_buf[sl]
            dst_buf[sl] = jnp.maximum(v, 0.0)


def _sc_edge_pass(a_tab, b_tab, ec, src, dst, zeros_nh):
    mesh = plsc.VectorSubcoreMesh(core_axis_name="c", subcore_axis_name="s")

    @functools.partial(
        pl.kernel,
        mesh=mesh,
        out_type=jax.ShapeDtypeStruct((_NC * N, H), jnp.float32),
        scratch_types=[
            pltpu.VMEM((_CHUNK,), jnp.int32),
            pltpu.VMEM((_CHUNK,), jnp.int32),
            pltpu.VMEM((_CHUNK, H), jnp.float32),
            pltpu.VMEM((_CHUNK, H), jnp.float32),
            pltpu.VMEM((_CHUNK, H), jnp.float32),
            pltpu.VMEM((_TAIL,), jnp.int32),
            pltpu.VMEM((_TAIL,), jnp.int32),
            pltpu.VMEM((_TAIL, H), jnp.float32),
            pltpu.VMEM((_TAIL, H), jnp.float32),
            pltpu.VMEM((_TAIL, H), jnp.float32),
            pltpu.VMEM_SHARED((N, H), jnp.float32),
            pltpu.SemaphoreType.DMA,
            pltpu.SemaphoreType.DMA,
        ],
    )
    def k(a_hbm, b_hbm, ec_hbm, src_hbm, dst_hbm, z_hbm, o_hbm,
          i1, i2, ra, rb, re, ti1, ti2, tra, trb, tre, acc, sem1, sem2):
        cid = lax.axis_index("c")
        sid = lax.axis_index("s")
        wid = sid * _NC + cid

        r0 = sid * _ROWS_PER_S
        pltpu.sync_copy(z_hbm.at[pl.ds(r0, _ROWS_PER_S)],
                        acc.at[pl.ds(r0, _ROWS_PER_S)])

        @pl.when(sid == _NS - 1)
        def _():
            pltpu.sync_copy(z_hbm.at[pl.ds(_NS * _ROWS_PER_S, _ROWS_REM)],
                            acc.at[pl.ds(_NS * _ROWS_PER_S, _ROWS_REM)])

        plsc.subcore_barrier()

        base = wid * _PER_W

        @pl.loop(0, _NFULL)
        def _(ci):
            off = base + ci * _CHUNK
            pltpu.sync_copy(src_hbm.at[pl.ds(off, _CHUNK)], i1)
            pltpu.sync_copy(dst_hbm.at[pl.ds(off, _CHUNK)], i2)
            c1 = pltpu.async_copy(a_hbm.at[i1], ra, sem1)
            c2 = pltpu.async_copy(b_hbm.at[i2], rb, sem2)
            pltpu.sync_copy(ec_hbm.at[pl.ds(off, _CHUNK)], re)
            c1.wait()
            c2.wait()
            _relu_add_rows(_CHUNK, ra, rb, re)
            pltpu.sync_copy(ra, acc.at[i2], add=True)

        off = base + _NFULL * _CHUNK
        pltpu.sync_copy(src_hbm.at[pl.ds(off, _TAIL)], ti1)
        pltpu.sync_copy(dst_hbm.at[pl.ds(off, _TAIL)], ti2)
        c1 = pltpu.async_copy(a_hbm.at[ti1], tra, sem1)
        c2 = pltpu.async_copy(b_hbm.at[ti2], trb, sem2)
        pltpu.sync_copy(ec_hbm.at[pl.ds(off, _TAIL)], tre)
        c1.wait()
        c2.wait()
        _relu_add_rows(_TAIL, tra, trb, tre)
        pltpu.sync_copy(tra, acc.at[ti2], add=True)

        plsc.subcore_barrier()
        pltpu.sync_copy(acc.at[pl.ds(r0, _ROWS_PER_S)],
                        o_hbm.at[pl.ds(cid * N + r0, _ROWS_PER_S)])

        @pl.when(sid == _NS - 1)
        def _():
            pltpu.sync_copy(
                acc.at[pl.ds(_NS * _ROWS_PER_S, _ROWS_REM)],
                o_hbm.at[pl.ds(cid * N + _NS * _ROWS_PER_S, _ROWS_REM)])

    out = k(a_tab, b_tab, ec, src, dst, zeros_nh)
    return out[:N], out[N:]


def _sc_degrees(dst, ones_hbm, zeros16_hbm):
    mesh = plsc.VectorSubcoreMesh(core_axis_name="c", subcore_axis_name="s")

    @functools.partial(
        pl.kernel,
        mesh=mesh,
        out_type=jax.ShapeDtypeStruct((_NC * N, _DEGW), jnp.float32),
        scratch_types=[
            pltpu.VMEM((_CHUNK,), jnp.int32),
            pltpu.VMEM((_CHUNK, _DEGW), jnp.float32),
            pltpu.VMEM((_TAIL,), jnp.int32),
            pltpu.VMEM((_TAIL, _DEGW), jnp.float32),
            pltpu.VMEM_SHARED((N, _DEGW), jnp.float32),
        ],
    )
    def k(d_hbm, one_hbm, z_hbm, o_hbm, idx, ones, tidx, tones, acc):
        cid = lax.axis_index("c")
        sid = lax.axis_index("s")
        wid = sid * _NC + cid

        r0 = sid * _ROWS_PER_S
        pltpu.sync_copy(z_hbm.at[pl.ds(r0, _ROWS_PER_S)],
                        acc.at[pl.ds(r0, _ROWS_PER_S)])

        @pl.when(sid == _NS - 1)
        def _():
            pltpu.sync_copy(z_hbm.at[pl.ds(_NS * _ROWS_PER_S, _ROWS_REM)],
                            acc.at[pl.ds(_NS * _ROWS_PER_S, _ROWS_REM)])

        pltpu.sync_copy(one_hbm, ones)
        pltpu.sync_copy(one_hbm.at[pl.ds(0, _TAIL)], tones)
        plsc.subcore_barrier()

        base = wid * _PER_W

        @pl.loop(0, _NFULL)
        def _(ci):
            off = base + ci * _CHUNK
            pltpu.sync_copy(d_hbm.at[pl.ds(off, _CHUNK)], idx)
            pltpu.sync_copy(ones, acc.at[idx], add=True)

        off = base + _NFULL * _CHUNK
        pltpu.sync_copy(d_hbm.at[pl.ds(off, _TAIL)], tidx)
        pltpu.sync_copy(tones, acc.at[tidx], add=True)

        plsc.subcore_barrier()
        pltpu.sync_copy(acc.at[pl.ds(r0, _ROWS_PER_S)],
                        o_hbm.at[pl.ds(cid * N + r0, _ROWS_PER_S)])

        @pl.when(sid == _NS - 1)
        def _():
            pltpu.sync_copy(
                acc.at[pl.ds(_NS * _ROWS_PER_S, _ROWS_REM)],
                o_hbm.at[pl.ds(cid * N + _NS * _ROWS_PER_S, _ROWS_REM)])

    out = k(dst, ones_hbm, zeros16_hbm)
    return out[:N, :1] + out[N:, :1]


def kernel(x, edge_attr, coords, bc_disp, bc_rot, edge_index, params):
    r2 = lambda v: v.reshape(1, -1)
    src = edge_index[0].astype(jnp.int32)
    dst = edge_index[1].astype(jnp.int32)

    (ne_w1, ne_b1), (ne_w2, ne_b2) = params['node_encoder']
    (ee_w1, ee_b1), (ee_w2, ee_b2) = params['edge_encoder']

    h = _node_encode(x, ne_w1, r2(ne_b1), ne_w2, r2(ne_b2))

    we_all = jnp.stack([lp['message_mlp'][0][0][2 * H:3 * H] for lp in params['mp_layers']])
    be_all = jnp.stack([r2(lp['message_mlp'][0][1]) for lp in params['mp_layers']])
    ecs = _edge_prep(edge_attr, ee_w1, r2(ee_b1), ee_w2, r2(ee_b2), we_all, be_all)

    zeros_nh = jnp.zeros((N, H), jnp.float32)
    ones_rows = jnp.ones((_CHUNK, _DEGW), jnp.float32)
    deg = _sc_degrees(dst, ones_rows, zeros_nh)

    for li, lp in enumerate(params['mp_layers']):
        (w1, _b1), (w2, b2) = lp['message_mlp']
        wa, wb, wr = w1[:H], w1[H:2 * H], w1[3 * H:]
        (wu1, bu1), (wu2, bu2) = lp['update_mlp']
        wh, wag = wu1[:H], wu1[H:]
        ec = ecs[li]
        for _ in range(2):
            a_tab, b_tab = _ab_tables(h, coords, wa, wb, wr)
            p0, p1 = _sc_edge_pass(a_tab, b_tab, ec, src, dst, zeros_nh)
            h = _update(h, p0, p1, deg, w2, r2(b2), wh, wag, r2(bu1),
                        wu2, r2(bu2), r2(lp['ln_g']), r2(lp['ln_b']))

    (d_w1, d_b1), (d_w2, d_b2), (d_w3, d_b3), (d_w4, d_b4) = params['decoder']
    return _decoder(coords, h, bc_disp, bc_rot,
                    d_w1[:3], d_w1[3:], r2(d_b1), d_w2, r2(d_b2),
                    d_w3, r2(d_b3), d_w4, r2(d_b4))


# speedup vs baseline: 3.7933x; 1.0528x over previous
import functools

import jax
import jax.numpy as jnp
from jax import lax
from jax.experimental import pallas as pl
from jax.experimental.pallas import tpu as pltpu
from jax.experimental.pallas import tpu_sc as plsc

N = 10000
E = 160000
H = 128

EB = 2000
N_EDGE_BLOCKS = E // EB


def _mlp2_body(x_ref, w1_ref, b1_ref, w2_ref, b2_ref, o_ref):
    t = jnp.dot(x_ref[...], w1_ref[...], preferred_element_type=jnp.float32)
    t = jax.nn.relu(t + b1_ref[...])
    o = jnp.dot(t, w2_ref[...], preferred_element_type=jnp.float32)
    o_ref[...] = o + b2_ref[...]


def _node_encode(x, w1, b1, w2, b2):
    return pl.pallas_call(
        _mlp2_body,
        out_shape=jax.ShapeDtypeStruct((N, H), jnp.float32),
    )(x, w1, b1, w2, b2)


def _edge_prep_body(ea_ref, w1_ref, b1_ref, w2_ref, b2_ref,
                    we_ref, be_ref, ec0_ref, ec1_ref, ec2_ref):
    t = jnp.dot(ea_ref[...], w1_ref[...], preferred_element_type=jnp.float32)
    t = jax.nn.relu(t + b1_ref[...])
    e = jnp.dot(t, w2_ref[...], preferred_element_type=jnp.float32) + b2_ref[...]
    for l, o_ref in enumerate((ec0_ref, ec1_ref, ec2_ref)):
        ec = jnp.dot(e, we_ref[l], preferred_element_type=jnp.float32)
        o_ref[...] = ec + be_ref[l]


def _edge_prep(edge_attr, w1, b1, w2, b2, we_all, be_all):
    blk = lambda i: (i, 0)
    zero = lambda i: (0, 0)
    return pl.pallas_call(
        _edge_prep_body,
        grid=(N_EDGE_BLOCKS,),
        in_specs=[
            pl.BlockSpec((EB, 10), blk),
            pl.BlockSpec((10, H), zero),
            pl.BlockSpec((1, H), zero),
            pl.BlockSpec((H, H), zero),
            pl.BlockSpec((1, H), zero),
            pl.BlockSpec((3, H, H), lambda i: (0, 0, 0)),
            pl.BlockSpec((3, 1, H), lambda i: (0, 0, 0)),
        ],
        out_specs=[pl.BlockSpec((EB, H), blk)] * 3,
        out_shape=[jax.ShapeDtypeStruct((E, H), jnp.float32)] * 3,
    )(edge_attr, w1, b1, w2, b2, we_all, be_all)


def _ab_body(h_ref, c_ref, wa_ref, wb_ref, wr_ref, a_ref, b_ref):
    cr = jnp.dot(c_ref[...], wr_ref[...], preferred_element_type=jnp.float32)
    a_ref[...] = jnp.dot(h_ref[...], wa_ref[...],
                         preferred_element_type=jnp.float32) - cr
    b_ref[...] = jnp.dot(h_ref[...], wb_ref[...],
                         preferred_element_type=jnp.float32) + cr


def _ab_tables(h, coords, wa, wb, wr):
    return pl.pallas_call(
        _ab_body,
        out_shape=[jax.ShapeDtypeStruct((N, H), jnp.float32)] * 2,
    )(h, coords, wa, wb, wr)


def _update_body(h_ref, p0_ref, p1_ref, deg_ref, w2_ref, b2_ref,
                 wh_ref, wa_ref, b1_ref, wu2_ref, bu2_ref,
                 g_ref, bln_ref, o_ref):
    h = h_ref[...]
    hsum = p0_ref[...] + p1_ref[...]
    aggr = jnp.dot(hsum, w2_ref[...], preferred_element_type=jnp.float32)
    aggr = aggr + deg_ref[...] * b2_ref[...]
    t = jnp.dot(h, wh_ref[...], preferred_element_type=jnp.float32)
    t = t + jnp.dot(aggr, wa_ref[...], preferred_element_type=jnp.float32)
    t = jax.nn.relu(t + b1_ref[...])
    out = jnp.dot(t, wu2_ref[...], preferred_element_type=jnp.float32) + bu2_ref[...]
    y = h + out
    mu = jnp.mean(y, axis=-1, keepdims=True)
    var = jnp.mean((y - mu) ** 2, axis=-1, keepdims=True)
    o_ref[...] = g_ref[...] * (y - mu) * lax.rsqrt(var + 1e-5) + bln_ref[...]


def _update(h, p0, p1, deg, w2, b2, wh, wa, b1, wu2, bu2, g, b):
    return pl.pallas_call(
        _update_body,
        out_shape=jax.ShapeDtypeStruct((N, H), jnp.float32),
    )(h, p0, p1, deg, w2, b2, wh, wa, b1, wu2, bu2, g, b)


def _decoder_body(c_ref, h_ref, bcd_ref, bcr_ref, wc_ref, wh_ref, b1_ref,
                  w2_ref, b2_ref, w3_ref, b3_ref, w4_ref, b4_ref, o_ref):
    c = c_ref[...]
    c_min = jnp.min(c, axis=0, keepdims=True)
    c_max = jnp.max(c, axis=0, keepdims=True)
    c_range = jnp.clip(c_max - c_min, 1e-8, None)
    cn = (c - c_min) / c_range
    t = jnp.dot(cn, wc_ref[...], preferred_element_type=jnp.float32)
    t = t + jnp.dot(h_ref[...], wh_ref[...], preferred_element_type=jnp.float32)
    t = jax.nn.relu(t + b1_ref[...])
    t = jnp.dot(t, w2_ref[...], preferred_element_type=jnp.float32) + b2_ref[...]
    t = jax.nn.relu(t)
    t = jnp.dot(t, w3_ref[...], preferred_element_type=jnp.float32) + b3_ref[...]
    t = jax.nn.relu(t)
    pred = jnp.dot(t, w4_ref[...], preferred_element_type=jnp.float32) + b4_ref[...]
    mask01 = 1.0 - bcd_ref[...]
    mask2 = 1.0 - bcr_ref[...]
    col = lax.broadcasted_iota(jnp.int32, pred.shape, 1)
    o_ref[...] = pred * jnp.where(col < 2, mask01, mask2)


def _decoder(coords, h, bc_disp, bc_rot, wc, wh, b1, w2, b2, w3, b3, w4, b4):
    return pl.pallas_call(
        _decoder_body,
        out_shape=jax.ShapeDtypeStruct((N, 3), jnp.float32),
    )(coords, h, bc_disp, bc_rot, wc, wh, b1, w2, b2, w3, b3, w4, b4)


_NC = 2
_NS = 16
_NW = _NC * _NS
_PER_W = E // _NW
_CHUNK = 48
_NFULL = _PER_W // _CHUNK
_TAIL = _PER_W - _NFULL * _CHUNK
_ROWS_PER_S = 624
_ROWS_REM = N - _NS * _ROWS_PER_S
_DEGW = H


def _relu_add_rows(n_rows, dst_buf, b_buf, e_buf):
    @pl.loop(0, n_rows)
    def _(r):
        for cc in range(H // 16):
            sl = (r, pl.ds(cc * 16, 16))
            v = dst_buf[sl] + b_buf[sl] + e_buf[sl]
            dst_buf[sl] = jnp.maximum(v, 0.0)


def _sc_edge_pass(a_tab, b_tab, ec, src, dst, zeros_nh):
    mesh = plsc.VectorSubcoreMesh(core_axis_name="c", subcore_axis_name="s")

    @functools.partial(
        pl.kernel,
        mesh=mesh,
        out_type=jax.ShapeDtypeStruct((_NC * N, H), jnp.float32),
        scratch_types=[
            pltpu.VMEM((_CHUNK,), jnp.int32),
            pltpu.VMEM((_CHUNK,), jnp.int32),
            pltpu.VMEM((_CHUNK,), jnp.int32),
            pltpu.VMEM((_CHUNK,), jnp.int32),
            pltpu.VMEM((_CHUNK, H), jnp.float32),
            pltpu.VMEM((_CHUNK, H), jnp.float32),
            pltpu.VMEM((_CHUNK, H), jnp.float32),
            pltpu.VMEM((_CHUNK, H), jnp.float32),
            pltpu.VMEM((_CHUNK, H), jnp.float32),
            pltpu.VMEM((_CHUNK, H), jnp.float32),
            pltpu.VMEM((_TAIL,), jnp.int32),
            pltpu.VMEM((_TAIL,), jnp.int32),
            pltpu.VMEM((_TAIL, H), jnp.float32),
            pltpu.VMEM((_TAIL, H), jnp.float32),
            pltpu.VMEM((_TAIL, H), jnp.float32),
            pltpu.VMEM_SHARED((N, H), jnp.float32),
            pltpu.SemaphoreType.DMA,
            pltpu.SemaphoreType.DMA,
            pltpu.SemaphoreType.DMA,
            pltpu.SemaphoreType.DMA,
        ],
    )
    def k(a_hbm, b_hbm, ec_hbm, src_hbm, dst_hbm, z_hbm, o_hbm,
          i1_0, i2_0, i1_1, i2_1, ra0, rb0, re0, ra1, rb1, re1,
          ti1, ti2, tra, trb, tre, acc,
          g0, g1, q0, q1):
        cid = lax.axis_index("c")
        sid = lax.axis_index("s")
        wid = sid * _NC + cid

        i1 = (i1_0, i1_1)
        i2 = (i2_0, i2_1)
        ra = (ra0, ra1)
        rb = (rb0, rb1)
        re = (re0, re1)
        gsem = (g0, g1)
        isem = (q0, q1)

        r0 = sid * _ROWS_PER_S
        pltpu.sync_copy(z_hbm.at[pl.ds(r0, _ROWS_PER_S)],
                        acc.at[pl.ds(r0, _ROWS_PER_S)])

        @pl.when(sid == _NS - 1)
        def _():
            pltpu.sync_copy(z_hbm.at[pl.ds(_NS * _ROWS_PER_S, _ROWS_REM)],
                            acc.at[pl.ds(_NS * _ROWS_PER_S, _ROWS_REM)])

        plsc.subcore_barrier()

        base = wid * _PER_W

        def pair(c0, prefetch):
            off0 = base + c0 * _CHUNK
            off1 = off0 + _CHUNK
            ga0 = pltpu.async_copy(a_hbm.at[i1[0]], ra[0], gsem[0])
            gb0 = pltpu.async_copy(b_hbm.at[i2[0]], rb[0], gsem[0])
            ge0 = pltpu.async_copy(ec_hbm.at[pl.ds(off0, _CHUNK)], re[0], gsem[0])
            ga1 = pltpu.async_copy(a_hbm.at[i1[1]], ra[1], gsem[1])
            gb1 = pltpu.async_copy(b_hbm.at[i2[1]], rb[1], gsem[1])
            ge1 = pltpu.async_copy(ec_hbm.at[pl.ds(off1, _CHUNK)], re[1], gsem[1])
            ga0.wait()
            gb0.wait()
            ge0.wait()
            _relu_add_rows(_CHUNK, ra[0], rb[0], re[0])
            pltpu.sync_copy(ra[0], acc.at[i2[0]], add=True)
            if prefetch:
                p10 = pltpu.async_copy(src_hbm.at[pl.ds(off0 + 2 * _CHUNK, _CHUNK)],
                                       i1[0], isem[0])
                p20 = pltpu.async_copy(dst_hbm.at[pl.ds(off0 + 2 * _CHUNK, _CHUNK)],
                                       i2[0], isem[0])
            ga1.wait()
            gb1.wait()
            ge1.wait()
            _relu_add_rows(_CHUNK, ra[1], rb[1], re[1])
            pltpu.sync_copy(ra[1], acc.at[i2[1]], add=True)
            if prefetch:
                p11 = pltpu.async_copy(src_hbm.at[pl.ds(off1 + 2 * _CHUNK, _CHUNK)],
                                       i1[1], isem[1])
                p21 = pltpu.async_copy(dst_hbm.at[pl.ds(off1 + 2 * _CHUNK, _CHUNK)],
                                       i2[1], isem[1])
                p10.wait()
                p20.wait()
                p11.wait()
                p21.wait()

        pltpu.sync_copy(src_hbm.at[pl.ds(base, _CHUNK)], i1[0])
        pltpu.sync_copy(dst_hbm.at[pl.ds(base, _CHUNK)], i2[0])
        pltpu.sync_copy(src_hbm.at[pl.ds(base + _CHUNK, _CHUNK)], i1[1])
        pltpu.sync_copy(dst_hbm.at[pl.ds(base + _CHUNK, _CHUNK)], i2[1])

        @pl.loop(0, _NFULL // 2 - 1)
        def _(kk):
            pair(2 * kk, prefetch=True)

        pair(_NFULL - 2, prefetch=False)

        off = base + _NFULL * _CHUNK
        pltpu.sync_copy(src_hbm.at[pl.ds(off, _TAIL)], ti1)
        pltpu.sync_copy(dst_hbm.at[pl.ds(off, _TAIL)], ti2)
        c1 = pltpu.async_copy(a_hbm.at[ti1], tra, gsem[0])
        c2 = pltpu.async_copy(b_hbm.at[ti2], trb, gsem[1])
        pltpu.sync_copy(ec_hbm.at[pl.ds(off, _TAIL)], tre)
        c1.wait()
        c2.wait()
        _relu_add_rows(_TAIL, tra, trb, tre)
        pltpu.sync_copy(tra, acc.at[ti2], add=True)

        plsc.subcore_barrier()
        pltpu.sync_copy(acc.at[pl.ds(r0, _ROWS_PER_S)],
                        o_hbm.at[pl.ds(cid * N + r0, _ROWS_PER_S)])

        @pl.when(sid == _NS - 1)
        def _():
            pltpu.sync_copy(
                acc.at[pl.ds(_NS * _ROWS_PER_S, _ROWS_REM)],
                o_hbm.at[pl.ds(cid * N + _NS * _ROWS_PER_S, _ROWS_REM)])

    out = k(a_tab, b_tab, ec, src, dst, zeros_nh)
    return out[:N], out[N:]


def _sc_degrees(dst, ones_hbm, zeros16_hbm):
    mesh = plsc.VectorSubcoreMesh(core_axis_name="c", subcore_axis_name="s")

    @functools.partial(
        pl.kernel,
        mesh=mesh,
        out_type=jax.ShapeDtypeStruct((_NC * N, _DEGW), jnp.float32),
        scratch_types=[
            pltpu.VMEM((_CHUNK,), jnp.int32),
            pltpu.VMEM((_CHUNK, _DEGW), jnp.float32),
            pltpu.VMEM((_TAIL,), jnp.int32),
            pltpu.VMEM((_TAIL, _DEGW), jnp.float32),
            pltpu.VMEM_SHARED((N, _DEGW), jnp.float32),
        ],
    )
    def k(d_hbm, one_hbm, z_hbm, o_hbm, idx, ones, tidx, tones, acc):
        cid = lax.axis_index("c")
        sid = lax.axis_index("s")
        wid = sid * _NC + cid

        r0 = sid * _ROWS_PER_S
        pltpu.sync_copy(z_hbm.at[pl.ds(r0, _ROWS_PER_S)],
                        acc.at[pl.ds(r0, _ROWS_PER_S)])

        @pl.when(sid == _NS - 1)
        def _():
            pltpu.sync_copy(z_hbm.at[pl.ds(_NS * _ROWS_PER_S, _ROWS_REM)],
                            acc.at[pl.ds(_NS * _ROWS_PER_S, _ROWS_REM)])

        pltpu.sync_copy(one_hbm, ones)
        pltpu.sync_copy(one_hbm.at[pl.ds(0, _TAIL)], tones)
        plsc.subcore_barrier()

        base = wid * _PER_W

        @pl.loop(0, _NFULL)
        def _(ci):
            off = base + ci * _CHUNK
            pltpu.sync_copy(d_hbm.at[pl.ds(off, _CHUNK)], idx)
            pltpu.sync_copy(ones, acc.at[idx], add=True)

        off = base + _NFULL * _CHUNK
        pltpu.sync_copy(d_hbm.at[pl.ds(off, _TAIL)], tidx)
        pltpu.sync_copy(tones, acc.at[tidx], add=True)

        plsc.subcore_barrier()
        pltpu.sync_copy(acc.at[pl.ds(r0, _ROWS_PER_S)],
                        o_hbm.at[pl.ds(cid * N + r0, _ROWS_PER_S)])

        @pl.when(sid == _NS - 1)
        def _():
            pltpu.sync_copy(
                acc.at[pl.ds(_NS * _ROWS_PER_S, _ROWS_REM)],
                o_hbm.at[pl.ds(cid * N + _NS * _ROWS_PER_S, _ROWS_REM)])

    out = k(dst, ones_hbm, zeros16_hbm)
    return out[:N, :1] + out[N:, :1]


def kernel(x, edge_attr, coords, bc_disp, bc_rot, edge_index, params):
    r2 = lambda v: v.reshape(1, -1)
    src = edge_index[0].astype(jnp.int32)
    dst = edge_index[1].astype(jnp.int32)

    (ne_w1, ne_b1), (ne_w2, ne_b2) = params['node_encoder']
    (ee_w1, ee_b1), (ee_w2, ee_b2) = params['edge_encoder']

    h = _node_encode(x, ne_w1, r2(ne_b1), ne_w2, r2(ne_b2))

    we_all = jnp.stack([lp['message_mlp'][0][0][2 * H:3 * H] for lp in params['mp_layers']])
    be_all = jnp.stack([r2(lp['message_mlp'][0][1]) for lp in params['mp_layers']])
    ecs = _edge_prep(edge_attr, ee_w1, r2(ee_b1), ee_w2, r2(ee_b2), we_all, be_all)

    zeros_nh = jnp.zeros((N, H), jnp.float32)
    ones_rows = jnp.ones((_CHUNK, _DEGW), jnp.float32)
    deg = _sc_degrees(dst, ones_rows, zeros_nh)

    for li, lp in enumerate(params['mp_layers']):
        (w1, _b1), (w2, b2) = lp['message_mlp']
        wa, wb, wr = w1[:H], w1[H:2 * H], w1[3 * H:]
        (wu1, bu1), (wu2, bu2) = lp['update_mlp']
        wh, wag = wu1[:H], wu1[H:]
        ec = ecs[li]
        for _ in range(2):
            a_tab, b_tab = _ab_tables(h, coords, wa, wb, wr)
            p0, p1 = _sc_edge_pass(a_tab, b_tab, ec, src, dst, zeros_nh)
            h = _update(h, p0, p1, deg, w2, r2(b2), wh, wag, r2(bu1),
                        wu2, r2(bu2), r2(lp['ln_g']), r2(lp['ln_b']))

    (d_w1, d_b1), (d_w2, d_b2), (d_w3, d_b3), (d_w4, d_b4) = params['decoder']
    return _decoder(coords, h, bc_disp, bc_rot,
                    d_w1[:3], d_w1[3:], r2(d_b1), d_w2, r2(d_b2),
                    d_w3, r2(d_b3), d_w4, r2(d_b4))
